# baseline (speedup 1.0000x reference)
import functools

import jax
import jax.numpy as jnp
from jax import lax
from jax.experimental import pallas as pl
from jax.experimental.pallas import tpu as pltpu
from jax.experimental.pallas import tpu_sc as plsc

_B, _T, _H = 1024, 200, 128
_NW = 32
_BPW = _B // _NW

_CHUNKS = ((0, 128), (128, 72))

_KW, _KB, _OW, _OB, _TW, _TB, _SW, _SB = 0, 8, 16, 24, 32, 33, 34, 38


def _sc_body(kidx0, kidx1, kidx2, oidx, sidx0, sidx1, kcf, ocf, scf,
             wpack, s_tab0, s_tab1, k_tab0, k_tab1, k_tab2, o_tab0,
             out_s, out_k, out_o,
             idxv, grows, crow, kcv, ocv, scv, wv,
             gsem, ssem, csem, s_sem):
    wid = lax.axis_index("s") * 2 + lax.axis_index("c")
    b0 = wid * _BPW

    pltpu.sync_copy(wpack, wv)

    pltpu.sync_copy(sidx0.at[pl.ds(b0, _BPW)], idxv.at[0, 0, pl.ds(0, _BPW)])
    pltpu.sync_copy(sidx1.at[pl.ds(b0, _BPW)], idxv.at[1, 0, pl.ds(0, _BPW)])
    pltpu.sync_copy(scf.at[pl.ds(b0 * 4, _BPW * 4)], scv)
    for j, tab in ((0, s_tab0), (1, s_tab1)):
        pltpu.async_copy(tab.at[idxv.at[j, 0, pl.ds(0, _BPW)]],
                         grows.at[j, pl.ds(0, _BPW)], s_sem).wait()
        pltpu.sync_copy(grows.at[j, pl.ds(0, _BPW)],
                        out_s.at[j, pl.ds(b0, _BPW), :])
    for f in range(4):
        w8 = [wv[_SW + f, pl.ds(g * 16, 16)] for g in range(8)]
        b8 = [wv[_SB + f, pl.ds(g * 16, 16)] for g in range(8)]

        @pl.loop(0, _BPW * 4 // 16)
        def _(v8):
            v = scv[pl.ds(v8 * 16, 16)]
            for q in range(4):
                x = v[q * 4 + f]
                for g in range(8):
                    crow[0, v8 * 4 + q, pl.ds(g * 16, 16)] = x * w8[g] + b8[g]
        pltpu.sync_copy(crow.at[0, pl.ds(0, _BPW)],
                        out_s.at[2 + f, pl.ds(b0, _BPW), :])

    cat = ((k_tab0, out_k, 0), (k_tab1, out_k, 1),
           (k_tab2, out_k, 2), (o_tab0, out_o, 0))
    cu = ([(kcv, out_k, 3 + f, f, _KW + f, _KB + f) for f in range(8)]
          + [(ocv, out_o, 1 + f, f, _OW + f, _OB + f) for f in range(8)])
    ncu = len(cu)

    def gissue(step):
        lk, (off, n) = step // 2, _CHUNKS[step % 2]
        pltpu.async_copy(cat[lk][0].at[idxv.at[lk, 0, pl.ds(off, n)]],
                         grows.at[step % 2, pl.ds(0, n)], gsem.at[step % 2])

    def gwait(step):
        lk, (off, n) = step // 2, _CHUNKS[step % 2]
        pltpu.make_async_copy(cat[lk][0].at[idxv.at[lk, 0, pl.ds(off, n)]],
                              grows.at[step % 2, pl.ds(0, n)],
                              gsem.at[step % 2]).wait()

    def scat(b, step, wait):
        lk, (off, n) = step // 2, _CHUNKS[step % 2]
        _, outref, slot = cat[lk]
        c = pltpu.make_async_copy(grows.at[step % 2, pl.ds(0, n)],
                                  outref.at[b, slot, pl.ds(off, n), :],
                                  ssem.at[step % 2])
        c.wait() if wait else c.start()

    def cdma(b, u, p, wait):
        src, outref, slot, f, wr, br = cu[u % ncu]
        c = pltpu.make_async_copy(crow.at[p], outref.at[b, slot, :, :],
                                  csem.at[p])
        c.wait() if wait else c.start()

    @pl.loop(0, _BPW)
    def _(bi):
        b = b0 + bi
        pltpu.sync_copy(kidx0.at[b], idxv.at[0])
        pltpu.sync_copy(kidx1.at[b], idxv.at[1])
        pltpu.sync_copy(kidx2.at[b], idxv.at[2])
        pltpu.sync_copy(oidx.at[b], idxv.at[3])
        pltpu.sync_copy(kcf.at[b], kcv)
        pltpu.sync_copy(ocf.at[b], ocv)

        @pl.when(bi > 0)
        def _():
            scat(b, 6, True)
        gissue(0)
        for i in range(8):
            gwait(i)
            scat(b, i, False)
            if i < 7:
                if i == 0:
                    @pl.when(bi > 0)
                    def _():
                        scat(b, 7, True)
                else:
                    scat(b, i - 1, True)
                gissue(i + 1)

        for u, (src, outref, slot, f, wr, br) in enumerate(cu):
            p = (bi * ncu + u) % 2
            if u >= 2:
                cdma(b, u - 2, p, True)
            else:
                @pl.when(bi > 0)
                def _():
                    cdma(b, u - 2, p, True)

            w8 = [wv[wr, pl.ds(g * 16, 16)] for g in range(8)]
            b8 = [wv[br, pl.ds(g * 16, 16)] for g in range(8)]

            @pl.loop(0, _T // 2)
            def _(tt):
                v = src[0, pl.ds(tt * 16, 16)]
                for h2 in range(2):
                    x = v[h2 * 8 + f]
                    for g in range(8):
                        crow[p, tt * 2 + h2, pl.ds(g * 16, 16)] = (
                            x * w8[g] + b8[g])
            cdma(b, u, p, False)

    bl = b0 + _BPW - 1
    for step in (6, 7):
        scat(bl, step, True)
    for u in (ncu - 2, ncu - 1):
        cdma(bl, u, ((_BPW - 1) * ncu + u) % 2, True)


_sc_all = functools.partial(
    pl.kernel,
    out_type=[
        jax.ShapeDtypeStruct((6, _B, _H), jnp.float32),
        jax.ShapeDtypeStruct((_B, 11, _T, _H), jnp.float32),
        jax.ShapeDtypeStruct((_B, 9, _T, _H), jnp.float32),
    ],
    mesh=plsc.VectorSubcoreMesh(core_axis_name="c", subcore_axis_name="s"),
    scratch_types=[
        pltpu.VMEM((4, 1, _T), jnp.int32),
        pltpu.VMEM((2, 128, _H), jnp.float32),
        pltpu.VMEM((2, _T, _H), jnp.float32),
        pltpu.VMEM((1, _T * 8), jnp.float32),
        pltpu.VMEM((1, _T * 8), jnp.float32),
        pltpu.VMEM((_BPW * 4,), jnp.float32),
        pltpu.VMEM((42, _H), jnp.float32),
        pltpu.SemaphoreType.DMA((2,)),
        pltpu.SemaphoreType.DMA((2,)),
        pltpu.SemaphoreType.DMA((2,)),
        pltpu.SemaphoreType.DMA,
    ],
)(_sc_body)


_TBB = 32


def _tc_tgt_body(tg, tw, tb, out):
    out[:, :, :] = tg[:, :, 0:1] * tw[0:1, :] + tb[0:1, :]


def _tc_tgt(tg, tw, tb):
    return pl.pallas_call(
        _tc_tgt_body,
        grid=(_B // _TBB,),
        in_specs=[
            pl.BlockSpec((_TBB, _T, 1), lambda i: (i, 0, 0)),
            pl.BlockSpec((1, _H), lambda i: (0, 0)),
            pl.BlockSpec((1, _H), lambda i: (0, 0)),
        ],
        out_specs=pl.BlockSpec((_TBB, _T, _H), lambda i: (i, 0, 0)),
        out_shape=jax.ShapeDtypeStruct((_B, _T, _H), jnp.float32),
    )(tg, tw, tb)


def kernel(s_cat, s_cont, k_cat, k_cont, o_cat, o_cont, target,
           s_tab0, s_tab1, k_tab0, k_tab1, k_tab2, o_tab0,
           s_cont_w, s_cont_b, k_cont_w, k_cont_b, o_cont_w, o_cont_b,
           tgt_w, tgt_b):
    i32, f32 = jnp.int32, jnp.float32
    kidx0 = k_cat[:, :, 0].astype(i32).reshape(_B, 1, _T)
    kidx1 = k_cat[:, :, 1].astype(i32).reshape(_B, 1, _T)
    kidx2 = k_cat[:, :, 2].astype(i32).reshape(_B, 1, _T)
    oidx = o_cat[:, :, 0].astype(i32).reshape(_B, 1, _T)
    sidx0 = s_cat[:, 0, 0].astype(i32)
    sidx1 = s_cat[:, 0, 1].astype(i32)
    kcf = k_cont.reshape(_B, 1, _T * 8)
    ocf = o_cont.reshape(_B, 1, _T * 8)
    scf = s_cont[:, 0, :].reshape(_B * 4)
    wpack = jnp.concatenate(
        [k_cont_w, k_cont_b, o_cont_w, o_cont_b, tgt_w, tgt_b,
         s_cont_w, s_cont_b], axis=0).astype(f32)

    out_s, out_k, out_o = _sc_all(
        kidx0, kidx1, kidx2, oidx, sidx0, sidx1, kcf, ocf, scf,
        wpack, s_tab0, s_tab1, k_tab0, k_tab1, k_tab2, o_tab0)
    out_t = _tc_tgt(target, tgt_w, tgt_b)
    return (jnp.swapaxes(out_s, 0, 1),
            jnp.swapaxes(out_k, 1, 2),
            jnp.swapaxes(out_o, 1, 2),
            out_t.reshape(_B, _T, 1, _H))
